# baseline (speedup 1.0000x reference)
import functools

import jax
import jax.numpy as jnp
from jax import lax
from jax.experimental import pallas as pl
from jax.experimental.pallas import tpu as pltpu
from jax.experimental.pallas import tpu_sc as plsc

B, S, D = 4, 4096, 1024
NC, NS = 2, 16
NW = NC * NS
ROWS_PER_W = S // NW
C = 32
NCHUNK = ROWS_PER_W // C
CHUNK = C * D
VEC = 16
UNROLL = 8

_mesh = plsc.VectorSubcoreMesh(core_axis_name="c", subcore_axis_name="s")


@functools.partial(
    pl.kernel,
    out_type=jax.ShapeDtypeStruct((B * S * D,), jnp.float32),
    mesh=_mesh,
    scratch_types=[
        pltpu.VMEM((CHUNK,), jnp.float32),
        pltpu.VMEM((CHUNK,), jnp.float32),
    ],
)
def _pos_add_kernel(x_hbm, emb_hbm, out_hbm, e_buf, o_buf):
    wid = lax.axis_index("s") * NC + lax.axis_index("c")
    s0 = wid * ROWS_PER_W

    def add_body(i, _):
        base = i * (VEC * UNROLL)
        for u in range(UNROLL):
            off = base + u * VEC
            e = e_buf[pl.ds(off, VEC)]
            plsc.addupdate(o_buf.at[pl.ds(off, VEC)], e)
        return 0

    for k in range(NCHUNK):
        sc = s0 + k * C
        pltpu.sync_copy(emb_hbm.at[pl.ds(sc * D, CHUNK)], e_buf)
        for b in range(B):
            row0 = b * S + sc
            pltpu.sync_copy(x_hbm.at[pl.ds(row0 * D, CHUNK)], o_buf)
            lax.fori_loop(0, CHUNK // (VEC * UNROLL), add_body, 0)
            pltpu.sync_copy(o_buf, out_hbm.at[pl.ds(row0 * D, CHUNK)])


def kernel(inputs, embeddings):
    x_flat = inputs.reshape(-1)
    emb_flat = embeddings.reshape(-1)
    out = _pos_add_kernel(x_flat, emb_flat)
    return out.reshape(inputs.shape)
